# baseline (speedup 1.0000x reference)
import functools

import jax
import jax.numpy as jnp
from jax import lax
from jax.experimental import pallas as pl
from jax.experimental.pallas import tpu as pltpu
from jax.experimental.pallas import tpu_sc as plsc

NUM_EMB = 1000000
D = 32
B = 4096
HIST = 200
HISTP = 201

NC = 2
NS = 16
NW = NC * NS
B_PER_W = B // NW
LT = HIST // 8
HC = 512
NPAIR = LT
STGP = 517


def _body(x_hbm, table_hbm, out_hbm, idx_v, gidx, rows0, rows1, stg0, stg1,
          gsem0, gsem1, osem0, osem1):
    wid = lax.axis_index("s") * NC + lax.axis_index("c")
    pltpu.sync_copy(x_hbm.at[wid], idx_v.at[:, pl.ds(0, HIST)])
    iot = lax.iota(jnp.int32, 16)
    bases = [k * 16 + iot for k in range(8)]
    dvec0 = iot * STGP
    dvec1 = dvec0 + 16 * STGP

    def rearrange(lt, _):
        for li in range(8):
            col = lt * 8 + li
            cvec = iot * 0 + col
            for k in range(8):
                vals = plsc.load_gather(idx_v, [bases[k], cvec])
                gidx[lt, pl.ds(li * 128 + k * 16, 16)] = vals
        return 0

    lax.fori_loop(0, LT, rearrange, 0)

    def issue_gathers(hc, rows_buf, sem):
        lt = hc >> 1
        off = (hc & 1) * HC
        for g in range(4):
            pltpu.async_copy(
                table_hbm.at[gidx.at[lt, pl.ds(off + g * 128, 128)]],
                rows_buf.at[pl.ds(g * 128, 128)],
                sem,
            )

    def drain_gathers(rows_buf, sem):
        pltpu.make_async_copy(
            table_hbm.at[pl.ds(0, HC)], rows_buf, sem
        ).wait()

    def issue_out(hc, stg_buf, sem):
        lt = hc >> 1
        off = (hc & 1) * HC
        pltpu.async_copy(
            stg_buf.at[:, pl.ds(0, HC)],
            out_hbm.at[:, lt, wid, pl.ds(off, HC)],
            sem,
        )

    def drain_out(stg_buf, sem):
        pltpu.make_async_copy(
            stg_buf.at[:, pl.ds(0, HC)],
            out_hbm.at[:, 0, 0, pl.ds(0, HC)],
            sem,
        ).wait()

    def transpose(rows_buf, stg_buf):
        @plsc.parallel_loop(0, HC, unroll=4)
        def _(g):
            gvec = iot * 0 + g
            v0 = rows_buf[g, pl.ds(0, 16)]
            plsc.store_scatter(stg_buf, [iot, gvec], v0)
            v1 = rows_buf[g, pl.ds(16, 16)]
            plsc.store_scatter(stg_buf, [iot + 16, gvec], v1)

    issue_gathers(0, rows0, gsem0)

    def pair(k, _):
        ha = 2 * k
        hb = ha + 1
        issue_gathers(hb, rows1, gsem1)
        drain_gathers(rows0, gsem0)

        @pl.when(k > 0)
        def _():
            drain_out(stg0, osem0)

        transpose(rows0, stg0)
        issue_out(ha, stg0, osem0)

        @pl.when(k < NPAIR - 1)
        def _():
            issue_gathers(ha + 2, rows0, gsem0)

        drain_gathers(rows1, gsem1)

        @pl.when(k > 0)
        def _():
            drain_out(stg1, osem1)

        transpose(rows1, stg1)
        issue_out(hb, stg1, osem1)
        return 0

    lax.fori_loop(0, NPAIR, pair, 0)
    drain_out(stg0, osem0)
    drain_out(stg1, osem1)


NBLK_FULL = 7812
TAIL = NUM_EMB - NBLK_FULL * 128
JMAX = 244


def _tr_body(tabt_hbm, out_hbm, src0, src1, stg0, stg1,
             gsem0, gsem1, osem0, osem1):
    wid = lax.axis_index("s") * NC + lax.axis_index("c")
    iot = lax.iota(jnp.int32, 16)

    def issue_in(bid, src_buf, sem):
        pltpu.async_copy(
            tabt_hbm.at[:, pl.ds(bid * 128, 128)], src_buf, sem
        )

    def drain_in(src_buf, sem):
        pltpu.make_async_copy(
            tabt_hbm.at[:, pl.ds(0, 128)], src_buf, sem
        ).wait()

    def issue_out(bid, stg_buf, sem):
        pltpu.async_copy(
            stg_buf.at[:, pl.ds(0, D)],
            out_hbm.at[pl.ds(bid * 128, 128)],
            sem,
        )

    def drain_out(stg_buf, sem):
        pltpu.make_async_copy(
            stg_buf.at[:, pl.ds(0, D)], out_hbm.at[pl.ds(0, 128)], sem
        ).wait()

    def transpose(src_buf, stg_buf, ngroups):
        @plsc.parallel_loop(0, D, unroll=4)
        def _(d):
            dvec = iot * 0 + d
            for k in range(ngroups):
                v = src_buf[d, pl.ds(k * 16, 16)]
                plsc.store_scatter(stg_buf, [k * 16 + iot, dvec], v)

    issue_in(wid, src0, gsem0)

    def pair(k, _):
        ja = 2 * k
        jb = ja + 1
        issue_in(wid + NW * jb, src1, gsem1)
        drain_in(src0, gsem0)

        @pl.when(k > 0)
        def _():
            drain_out(stg0, osem0)

        transpose(src0, stg0, 8)
        issue_out(wid + NW * ja, stg0, osem0)

        @pl.when(k < JMAX // 2 - 1)
        def _():
            issue_in(wid + NW * (ja + 2), src0, gsem0)

        drain_in(src1, gsem1)

        @pl.when(k > 0)
        def _():
            drain_out(stg1, osem1)

        transpose(src1, stg1, 8)
        issue_out(wid + NW * jb, stg1, osem1)
        return 0

    lax.fori_loop(0, JMAX // 2, pair, 0)
    drain_out(stg0, osem0)
    drain_out(stg1, osem1)

    @pl.when(wid < 4)
    def _():
        bid = NW * JMAX + wid
        pltpu.sync_copy(tabt_hbm.at[:, pl.ds(bid * 128, 128)], src0)
        transpose(src0, stg0, 8)
        pltpu.sync_copy(
            stg0.at[:, pl.ds(0, D)], out_hbm.at[pl.ds(bid * 128, 128)]
        )

    @pl.when(wid == 4)
    def _():
        base = NBLK_FULL * 128
        pltpu.sync_copy(
            tabt_hbm.at[:, pl.ds(base, TAIL)], src0.at[:, pl.ds(0, TAIL)]
        )
        transpose(src0, stg0, TAIL // 16)
        pltpu.sync_copy(
            stg0.at[pl.ds(0, TAIL), pl.ds(0, D)],
            out_hbm.at[pl.ds(base, TAIL)],
        )


@functools.partial(
    pl.kernel,
    mesh=plsc.VectorSubcoreMesh(core_axis_name="c", subcore_axis_name="s"),
    compiler_params=pltpu.CompilerParams(
        use_tc_tiling_on_sc=False, needs_layout_passes=False
    ),
    out_type=jax.ShapeDtypeStruct((NUM_EMB, D), jnp.float32),
    scratch_types=[
        pltpu.VMEM((D, 128), jnp.float32),
        pltpu.VMEM((D, 128), jnp.float32),
        pltpu.VMEM((128, 33), jnp.float32),
        pltpu.VMEM((128, 33), jnp.float32),
        pltpu.SemaphoreType.DMA,
        pltpu.SemaphoreType.DMA,
        pltpu.SemaphoreType.DMA,
        pltpu.SemaphoreType.DMA,
    ],
)
def _tr_kernel(tabt_hbm, out_hbm, src0, src1, stg0, stg1,
               gsem0, gsem1, osem0, osem1):
    _tr_body(tabt_hbm, out_hbm, src0, src1, stg0, stg1,
             gsem0, gsem1, osem0, osem1)


@functools.partial(
    pl.kernel,
    mesh=plsc.VectorSubcoreMesh(core_axis_name="c", subcore_axis_name="s"),
    compiler_params=pltpu.CompilerParams(
        use_tc_tiling_on_sc=False, needs_layout_passes=False
    ),
    out_type=jax.ShapeDtypeStruct((D, LT, NW, 1024), jnp.float32),
    scratch_types=[
        pltpu.VMEM((B_PER_W, HISTP), jnp.int32),
        pltpu.VMEM((LT, 1024), jnp.int32),
        pltpu.VMEM((HC, D), jnp.float32),
        pltpu.VMEM((HC, D), jnp.float32),
        pltpu.VMEM((D, STGP), jnp.float32),
        pltpu.VMEM((D, STGP), jnp.float32),
        pltpu.SemaphoreType.DMA,
        pltpu.SemaphoreType.DMA,
        pltpu.SemaphoreType.DMA,
        pltpu.SemaphoreType.DMA,
    ],
)
def _emb_kernel(x_hbm, table_hbm, out_hbm, idx_v, gidx, rows0, rows1,
                stg0, stg1, gsem0, gsem1, osem0, osem1):
    _body(x_hbm, table_hbm, out_hbm, idx_v, gidx, rows0, rows1, stg0, stg1,
          gsem0, gsem1, osem0, osem1)


def kernel(x, table):
    x_r = x.astype(jnp.int32).reshape(NW, B_PER_W, HIST)
    table_rm = _tr_kernel(table.T)
    out4 = _emb_kernel(x_r, table_rm)
    out5 = out4.reshape(D, LT, NW, 8, 128)
    return out5.transpose(2, 4, 0, 1, 3).reshape(B, D, HIST)


# speedup vs baseline: 6.9203x; 6.9203x over previous
import functools

import jax
import jax.numpy as jnp
from jax import lax
from jax.experimental import pallas as pl
from jax.experimental.pallas import tpu as pltpu
from jax.experimental.pallas import tpu_sc as plsc

NUM_EMB = 1000000
D = 32
B = 4096
HIST = 200
HISTP = 201

NC = 2
NS = 16
NW = NC * NS
B_PER_W = B // NW
LT = HIST // 8
HC = 512
NPAIR = LT
STGP = 517


def _body(x_hbm, table_hbm, out_hbm, idx_v, gidx, rows0, rows1, stg0, stg1,
          gsem0, gsem1, osem0, osem1):
    wid = lax.axis_index("s") * NC + lax.axis_index("c")
    pltpu.sync_copy(x_hbm.at[wid], idx_v.at[:, pl.ds(0, HIST)])
    iot = lax.iota(jnp.int32, 16)
    bases = [k * 16 + iot for k in range(8)]
    dvec0 = iot * STGP
    dvec1 = dvec0 + 16 * STGP

    def rearrange(lt, _):
        for li in range(8):
            col = lt * 8 + li
            cvec = iot * 0 + col
            for k in range(8):
                vals = plsc.load_gather(idx_v, [bases[k], cvec])
                gidx[lt, pl.ds(li * 128 + k * 16, 16)] = vals
        return 0

    lax.fori_loop(0, LT, rearrange, 0)

    def issue_gathers(hc, rows_buf, sem):
        lt = hc >> 1
        off = (hc & 1) * HC
        for g in range(4):
            pltpu.async_copy(
                table_hbm.at[gidx.at[lt, pl.ds(off + g * 128, 128)]],
                rows_buf.at[pl.ds(g * 128, 128)],
                sem,
            )

    def drain_gathers(rows_buf, sem):
        pltpu.make_async_copy(
            table_hbm.at[pl.ds(0, HC)], rows_buf, sem
        ).wait()

    def issue_out(hc, stg_buf, sem):
        lt = hc >> 1
        off = (hc & 1) * HC
        pltpu.async_copy(
            stg_buf.at[:, pl.ds(0, HC)],
            out_hbm.at[:, lt, wid, pl.ds(off, HC)],
            sem,
        )

    def drain_out(stg_buf, sem):
        pltpu.make_async_copy(
            stg_buf.at[:, pl.ds(0, HC)],
            out_hbm.at[:, 0, 0, pl.ds(0, HC)],
            sem,
        ).wait()

    def transpose(rows_buf, stg_buf):
        @plsc.parallel_loop(0, HC, unroll=4)
        def _(g):
            gvec = iot * 0 + g
            v0 = rows_buf[g, pl.ds(0, 16)]
            plsc.store_scatter(stg_buf, [iot, gvec], v0)
            v1 = rows_buf[g, pl.ds(16, 16)]
            plsc.store_scatter(stg_buf, [iot + 16, gvec], v1)

    issue_gathers(0, rows0, gsem0)

    def pair(k, _):
        ha = 2 * k
        hb = ha + 1
        issue_gathers(hb, rows1, gsem1)
        drain_gathers(rows0, gsem0)

        @pl.when(k > 0)
        def _():
            drain_out(stg0, osem0)

        transpose(rows0, stg0)
        issue_out(ha, stg0, osem0)

        @pl.when(k < NPAIR - 1)
        def _():
            issue_gathers(ha + 2, rows0, gsem0)

        drain_gathers(rows1, gsem1)

        @pl.when(k > 0)
        def _():
            drain_out(stg1, osem1)

        transpose(rows1, stg1)
        issue_out(hb, stg1, osem1)
        return 0

    lax.fori_loop(0, NPAIR, pair, 0)
    drain_out(stg0, osem0)
    drain_out(stg1, osem1)


NBLK_FULL = 7812
TAIL = NUM_EMB - NBLK_FULL * 128
NPAD = NBLK_FULL * 128 + 128
JMAX = 244


def _tr_body(tabt_hbm, out_hbm, src0, src1, stg0, stg1,
             gsem0, gsem1, osem0, osem1):
    wid = lax.axis_index("s") * NC + lax.axis_index("c")
    iot = lax.iota(jnp.int32, 16)

    def issue_in(bid, src_buf, sem):
        pltpu.async_copy(tabt_hbm.at[:, bid], src_buf, sem)

    def drain_in(src_buf, sem):
        pltpu.make_async_copy(
            tabt_hbm.at[:, 0], src_buf, sem
        ).wait()

    def issue_out(bid, stg_buf, sem):
        pltpu.async_copy(
            stg_buf.at[:, pl.ds(0, D)],
            out_hbm.at[pl.ds(bid * 128, 128)],
            sem,
        )

    def drain_out(stg_buf, sem):
        pltpu.make_async_copy(
            stg_buf.at[:, pl.ds(0, D)], out_hbm.at[pl.ds(0, 128)], sem
        ).wait()

    def transpose(src_buf, stg_buf):
        @plsc.parallel_loop(0, D, unroll=4)
        def _(d):
            dvec = iot * 0 + d
            dh = d >> 3
            dl = d & 7
            for k in range(8):
                v = src_buf[dh, dl, pl.ds(k * 16, 16)]
                plsc.store_scatter(stg_buf, [k * 16 + iot, dvec], v)

    issue_in(wid, src0, gsem0)

    def pair(k, _):
        ja = 2 * k
        jb = ja + 1
        issue_in(wid + NW * jb, src1, gsem1)
        drain_in(src0, gsem0)

        @pl.when(k > 0)
        def _():
            drain_out(stg0, osem0)

        transpose(src0, stg0)
        issue_out(wid + NW * ja, stg0, osem0)

        @pl.when(k < JMAX // 2 - 1)
        def _():
            issue_in(wid + NW * (ja + 2), src0, gsem0)

        drain_in(src1, gsem1)

        @pl.when(k > 0)
        def _():
            drain_out(stg1, osem1)

        transpose(src1, stg1)
        issue_out(wid + NW * jb, stg1, osem1)
        return 0

    lax.fori_loop(0, JMAX // 2, pair, 0)
    drain_out(stg0, osem0)
    drain_out(stg1, osem1)

    @pl.when(wid < 4)
    def _():
        bid = NW * JMAX + wid
        pltpu.sync_copy(tabt_hbm.at[:, bid], src0)
        transpose(src0, stg0)
        pltpu.sync_copy(
            stg0.at[:, pl.ds(0, D)], out_hbm.at[pl.ds(bid * 128, 128)]
        )

    @pl.when(wid == 4)
    def _():
        pltpu.sync_copy(tabt_hbm.at[:, NBLK_FULL], src0)
        transpose(src0, stg0)
        pltpu.sync_copy(
            stg0.at[pl.ds(0, TAIL), pl.ds(0, D)],
            out_hbm.at[pl.ds(NBLK_FULL * 128, TAIL)],
        )


@functools.partial(
    pl.kernel,
    mesh=plsc.VectorSubcoreMesh(core_axis_name="c", subcore_axis_name="s"),
    compiler_params=pltpu.CompilerParams(
        use_tc_tiling_on_sc=False, needs_layout_passes=False
    ),
    out_type=jax.ShapeDtypeStruct((NUM_EMB, D), jnp.float32),
    scratch_types=[
        pltpu.VMEM((4, 8, 128), jnp.float32),
        pltpu.VMEM((4, 8, 128), jnp.float32),
        pltpu.VMEM((128, 33), jnp.float32),
        pltpu.VMEM((128, 33), jnp.float32),
        pltpu.SemaphoreType.DMA,
        pltpu.SemaphoreType.DMA,
        pltpu.SemaphoreType.DMA,
        pltpu.SemaphoreType.DMA,
    ],
)
def _tr_kernel(tabt_hbm, out_hbm, src0, src1, stg0, stg1,
               gsem0, gsem1, osem0, osem1):
    _tr_body(tabt_hbm, out_hbm, src0, src1, stg0, stg1,
             gsem0, gsem1, osem0, osem1)


@functools.partial(
    pl.kernel,
    mesh=plsc.VectorSubcoreMesh(core_axis_name="c", subcore_axis_name="s"),
    compiler_params=pltpu.CompilerParams(
        use_tc_tiling_on_sc=False, needs_layout_passes=False
    ),
    out_type=jax.ShapeDtypeStruct((D, LT, NW, 1024), jnp.float32),
    scratch_types=[
        pltpu.VMEM((B_PER_W, HISTP), jnp.int32),
        pltpu.VMEM((LT, 1024), jnp.int32),
        pltpu.VMEM((HC, D), jnp.float32),
        pltpu.VMEM((HC, D), jnp.float32),
        pltpu.VMEM((D, STGP), jnp.float32),
        pltpu.VMEM((D, STGP), jnp.float32),
        pltpu.SemaphoreType.DMA,
        pltpu.SemaphoreType.DMA,
        pltpu.SemaphoreType.DMA,
        pltpu.SemaphoreType.DMA,
    ],
)
def _emb_kernel(x_hbm, table_hbm, out_hbm, idx_v, gidx, rows0, rows1,
                stg0, stg1, gsem0, gsem1, osem0, osem1):
    _body(x_hbm, table_hbm, out_hbm, idx_v, gidx, rows0, rows1, stg0, stg1,
          gsem0, gsem1, osem0, osem1)


def kernel(x, table):
    x_r = x.astype(jnp.int32).reshape(NW, B_PER_W, HIST)
    t4 = (
        jnp.pad(table, ((0, NPAD - NUM_EMB), (0, 0)))
        .T.reshape(4, 8, NBLK_FULL + 1, 128)
        .transpose(0, 2, 1, 3)
    )
    table_rm = _tr_kernel(t4)
    out4 = _emb_kernel(x_r, table_rm)
    out5 = out4.reshape(D, LT, NW, 8, 128)
    return out5.transpose(2, 4, 0, 1, 3).reshape(B, D, HIST)
